# Initial kernel scaffold; baseline (speedup 1.0000x reference)
import jax
import jax.numpy as jnp
from jax.experimental import pallas as pl


def kernel(x, table, W, b):
    raise NotImplementedError("write your pallas kernel here")


# speedup vs baseline: 4.5738x; 4.5738x over previous
import functools

import jax
import jax.numpy as jnp
from jax import lax
from jax.experimental import pallas as pl
from jax.experimental.pallas import tpu as pltpu
from jax.experimental.pallas import tpu_sc as plsc

_NUM_CORES = 2
_NUM_SUBCORES = 16
_NW = _NUM_CORES * _NUM_SUBCORES


def _project_table(table, W, b2d, block_rows):
    num_emb, emb_dim = table.shape
    proj_dim = W.shape[0]

    def body(t_ref, w_ref, b_ref, o_ref):
        o_ref[...] = lax.dot_general(
            t_ref[...], w_ref[...],
            dimension_numbers=(((1,), (1,)), ((), ())),
            preferred_element_type=jnp.float32,
        ) + b_ref[...]

    return pl.pallas_call(
        body,
        grid=(num_emb // block_rows,),
        in_specs=[
            pl.BlockSpec((block_rows, emb_dim), lambda i: (i, 0)),
            pl.BlockSpec((proj_dim, emb_dim), lambda i: (0, 0)),
            pl.BlockSpec((1, proj_dim), lambda i: (0, 0)),
        ],
        out_specs=pl.BlockSpec((block_rows, proj_dim), lambda i: (i, 0)),
        out_shape=jax.ShapeDtypeStruct((num_emb, proj_dim), jnp.float32),
    )(table, W, b2d)


def _make_gather(n_idx, proj_dim, chunk):
    per_w = n_idx // _NW
    n_chunks = per_w // chunk
    mesh = plsc.VectorSubcoreMesh(core_axis_name="c", subcore_axis_name="s")

    @functools.partial(
        pl.kernel,
        mesh=mesh,
        out_type=jax.ShapeDtypeStruct((n_idx, proj_dim), jnp.float32),
        scratch_types=[
            pltpu.VMEM((chunk,), jnp.int32),
            pltpu.VMEM((chunk, proj_dim), jnp.float32),
            pltpu.SemaphoreType.DMA,
        ],
    )
    def gather_kernel(tab_hbm, idx_hbm, out_hbm, idx_v, rows_v, sem):
        wid = lax.axis_index("s") * _NUM_CORES + lax.axis_index("c")
        base = wid * per_w

        @pl.loop(0, n_chunks)
        def _(c):
            off = base + c * chunk
            pltpu.sync_copy(idx_hbm.at[pl.ds(off, chunk)], idx_v)
            pltpu.async_copy(tab_hbm.at[idx_v], rows_v, sem).wait()
            pltpu.sync_copy(rows_v, out_hbm.at[pl.ds(off, chunk)])

    return gather_kernel


def kernel(x, table, W, b):
    bsz, seq = x.shape
    proj_dim = W.shape[0]
    idx = x.reshape(-1).astype(jnp.int32)

    proj_table = _project_table(table, W, b.reshape(1, -1), block_rows=1000)

    n_idx = bsz * seq
    gather_kernel = _make_gather(n_idx, proj_dim, chunk=400)
    out = gather_kernel(proj_table, idx)
    return out.reshape(bsz, seq, proj_dim)


# speedup vs baseline: 6.5988x; 1.4428x over previous
import functools

import jax
import jax.numpy as jnp
from jax import lax
from jax.experimental import pallas as pl
from jax.experimental.pallas import tpu as pltpu
from jax.experimental.pallas import tpu_sc as plsc

_NUM_CORES = 2
_NUM_SUBCORES = 16
_NW = _NUM_CORES * _NUM_SUBCORES


def _project_table(table, W, b2d, block_rows):
    num_emb, emb_dim = table.shape
    proj_dim = W.shape[0]

    def body(t_ref, w_ref, b_ref, o_ref):
        o_ref[...] = lax.dot_general(
            t_ref[...], w_ref[...],
            dimension_numbers=(((1,), (1,)), ((), ())),
            preferred_element_type=jnp.float32,
        ) + b_ref[...]

    return pl.pallas_call(
        body,
        grid=(num_emb // block_rows,),
        in_specs=[
            pl.BlockSpec((block_rows, emb_dim), lambda i: (i, 0)),
            pl.BlockSpec((proj_dim, emb_dim), lambda i: (0, 0)),
            pl.BlockSpec((1, proj_dim), lambda i: (0, 0)),
        ],
        out_specs=pl.BlockSpec((block_rows, proj_dim), lambda i: (i, 0)),
        out_shape=jax.ShapeDtypeStruct((num_emb, proj_dim), jnp.float32),
    )(table, W, b2d)


def _make_gather(n_idx, proj_dim, chunk):
    per_w = n_idx // _NW
    n_chunks = per_w // chunk
    assert n_chunks % 2 == 0
    mesh = plsc.VectorSubcoreMesh(core_axis_name="c", subcore_axis_name="s")

    @functools.partial(
        pl.kernel,
        mesh=mesh,
        out_type=jax.ShapeDtypeStruct((n_idx, proj_dim), jnp.float32),
        scratch_types=[
            pltpu.VMEM((per_w,), jnp.int32),
            pltpu.VMEM((chunk, proj_dim), jnp.float32),
            pltpu.VMEM((chunk, proj_dim), jnp.float32),
            pltpu.SemaphoreType.DMA,
            pltpu.SemaphoreType.DMA,
            pltpu.SemaphoreType.DMA,
            pltpu.SemaphoreType.DMA,
        ],
    )
    def gather_kernel(tab_hbm, idx_hbm, out_hbm, idx_v, rows0, rows1,
                      g0, g1, o0, o1):
        wid = lax.axis_index("s") * _NUM_CORES + lax.axis_index("c")
        base = wid * per_w
        rows = (rows0, rows1)
        gsem = (g0, g1)
        osem = (o0, o1)
        pltpu.sync_copy(idx_hbm.at[pl.ds(base, per_w)], idx_v)

        def g_start(c, buf):
            pltpu.make_async_copy(
                tab_hbm.at[idx_v.at[pl.ds(c * chunk, chunk)]],
                rows[buf], gsem[buf]).start()

        def g_wait(buf):
            pltpu.make_async_copy(
                tab_hbm.at[idx_v.at[pl.ds(0, chunk)]],
                rows[buf], gsem[buf]).wait()

        def o_start(c, buf):
            pltpu.make_async_copy(
                rows[buf], out_hbm.at[pl.ds(base + c * chunk, chunk)],
                osem[buf]).start()

        def o_wait(buf):
            pltpu.make_async_copy(
                rows[buf], out_hbm.at[pl.ds(base, chunk)],
                osem[buf]).wait()

        g_start(0, 0)

        @pl.loop(0, n_chunks, step=2)
        def _(c):
            @pl.when(c > 0)
            def _():
                o_wait(1)
            g_start(c + 1, 1)
            g_wait(0)
            o_start(c, 0)

            @pl.when(c + 2 < n_chunks)
            def _():
                o_wait(0)
                g_start(c + 2, 0)
            g_wait(1)
            o_start(c + 1, 1)

        o_wait(0)
        o_wait(1)

    return gather_kernel


def kernel(x, table, W, b):
    bsz, seq = x.shape
    proj_dim = W.shape[0]
    idx = x.reshape(-1).astype(jnp.int32)

    proj_table = _project_table(table, W, b.reshape(1, -1), block_rows=4000)

    n_idx = bsz * seq
    gather_kernel = _make_gather(n_idx, proj_dim, chunk=400)
    out = gather_kernel(proj_table, idx)
    return out.reshape(bsz, seq, proj_dim)
